# baseline (speedup 1.0000x reference)
import functools

import jax
import jax.numpy as jnp
from jax import lax
from jax.experimental import pallas as pl
from jax.experimental.pallas import tpu as pltpu
from jax.experimental.pallas import tpu_sc as plsc

N = 10000
E = 160000
D = 256
DE = 16
DH = D // 2
NSC = 2
NSUB = 16
EPS = E // NSUB
K = 80
KH = K // 2
NCHUNK = EPS // K
RPS = 624
LANES = 16


def _edge_mlp_block(ea_ref, we_ref, h_ref, out_ref, h2_ref):
    z = jax.nn.relu(lax.dot_general(
        ea_ref[...], we_ref[...],
        dimension_numbers=(((0,), (0,)), ((), ())),
        preferred_element_type=jnp.float32))
    ss = jnp.sum(z * z, axis=1, keepdims=True)
    inv = jnp.where(ss == 0.0, 1.0, lax.rsqrt(ss))
    z = z * inv
    out_ref[0] = z[:, :DH]
    out_ref[1] = z[:, DH:]
    h2_ref[0] = h_ref[:, :DH]
    h2_ref[1] = h_ref[:, DH:]


def _edge_mlp(edge_attr_t, W_edge, h_neigh):
    BE = 3200
    BN = 200
    return pl.pallas_call(
        _edge_mlp_block,
        grid=(E // BE,),
        in_specs=[
            pl.BlockSpec((DE, BE), lambda i: (0, i)),
            pl.BlockSpec((DE, D), lambda i: (0, 0)),
            pl.BlockSpec((BN, D), lambda i: (i, 0)),
        ],
        out_specs=[
            pl.BlockSpec((NSC, BE, DH), lambda i: (0, i, 0)),
            pl.BlockSpec((NSC, BN, DH), lambda i: (0, i, 0)),
        ],
        out_shape=[
            jax.ShapeDtypeStruct((NSC, E, DH), jnp.float32),
            jax.ShapeDtypeStruct((NSC, N, DH), jnp.float32),
        ],
    )(edge_attr_t, W_edge, h_neigh)


def _combine_block(hs_ref, ws_ref, wn_ref, agg_ref, out_ref):
    agg = jnp.concatenate([agg_ref[0], agg_ref[1]], axis=1)
    z = jnp.dot(hs_ref[...], ws_ref[...], preferred_element_type=jnp.float32)
    z = z + jnp.dot(agg, wn_ref[...], preferred_element_type=jnp.float32)
    z = jax.nn.relu(z)
    ss = jnp.sum(z * z, axis=1, keepdims=True)
    inv = jnp.where(ss == 0.0, 1.0, lax.rsqrt(ss))
    out_ref[...] = z * inv


def _combine(h_self, W_self, W_neigh, agg2):
    BN = 1000
    return pl.pallas_call(
        _combine_block,
        grid=(N // BN,),
        in_specs=[
            pl.BlockSpec((BN, D), lambda i: (i, 0)),
            pl.BlockSpec((D, D), lambda i: (0, 0)),
            pl.BlockSpec((D, D), lambda i: (0, 0)),
            pl.BlockSpec((NSC, BN, DH), lambda i: (0, i, 0)),
        ],
        out_specs=pl.BlockSpec((BN, D), lambda i: (i, 0)),
        out_shape=jax.ShapeDtypeStruct((N, D), jnp.float32),
    )(h_self, W_self, W_neigh, agg2)


def _sc_body(eh_hbm, src_hbm, dst_hbm, h2_hbm, out_hbm,
             agg_sh, src_v, dstb, hbuf, ehbuf,
             sem_e, sem_g0, sem_g1, sem_g2,
             sem_s0, sem_s1, sem_s2, sem_d0, sem_d1):
    c = lax.axis_index("core")
    s = lax.axis_index("subcore")

    @pl.loop(0, K)
    def _(i):
        for q in range(DH // LANES):
            ehbuf[i, pl.ds(q * LANES, LANES)] = jnp.zeros((LANES,), jnp.float32)

    @pl.loop(0, 7)
    def _(r):
        pltpu.sync_copy(ehbuf, agg_sh.at[pl.ds(s * RPS + r * K, K)])

    @pl.when(s < NSUB - 1)
    def _():
        pltpu.sync_copy(ehbuf.at[pl.ds(0, 64)],
                        agg_sh.at[pl.ds(s * RPS + 7 * K, 64)])

    @pl.when(s == NSUB - 1)
    def _():
        pltpu.sync_copy(ehbuf, agg_sh.at[pl.ds(s * RPS + 7 * K, K)])

    plsc.subcore_barrier()

    pltpu.sync_copy(src_hbm.at[s, pl.ds(2, NCHUNK - 2)], src_v)

    gsems = (sem_g0, sem_g1, sem_g2)
    ssems = (sem_s0, sem_s1, sem_s2)
    dsems = (sem_d0, sem_d1)

    def eh_src(j):
        return eh_hbm.at[c, pl.ds(s * EPS + j * K, K)]

    def gather_src(j):
        return h2_hbm.at[c].at[src_v.at[j - 2]]

    def dst_src(j):
        return dst_hbm.at[s, j]

    pltpu.sync_copy(src_hbm.at[s, 0], dstb.at[0])
    pltpu.sync_copy(src_hbm.at[s, 1], dstb.at[1])
    pltpu.async_copy(eh_src(0), ehbuf, sem_e)
    pltpu.async_copy(h2_hbm.at[c].at[dstb.at[0]], hbuf.at[0], sem_g0)
    pltpu.async_copy(h2_hbm.at[c].at[dstb.at[1]], hbuf.at[1], sem_g1)

    def step(j, b, p, wait_scat, dst_sync, issue_eh, issue_dst,
             issue_gather):
        b2 = (b + 2) % 3
        if dst_sync:
            pltpu.make_async_copy(h2_hbm.at[c].at[dstb.at[p]],
                                  hbuf.at[b], gsems[b]).wait()
            pltpu.sync_copy(dst_src(j), dstb.at[p])
        else:
            pltpu.make_async_copy(gather_src(j), hbuf.at[b],
                                  gsems[b]).wait()
        pltpu.make_async_copy(eh_src(j), ehbuf, sem_e).wait()

        hb = hbuf.at[b]

        @pl.loop(0, K)
        def _(i):
            for q in range(DH // LANES):
                sl = (i, pl.ds(q * LANES, LANES))
                hb[sl] = hb[sl] * ehbuf[sl]

        if issue_eh:
            pltpu.async_copy(eh_src(j + 1), ehbuf, sem_e)

        if not dst_sync:
            pltpu.make_async_copy(dst_src(j), dstb.at[p], dsems[p]).wait()
        pltpu.async_copy(hbuf.at[b], agg_sh.at[dstb.at[p]], ssems[b],
                         add=True)

        if wait_scat:
            pltpu.make_async_copy(hbuf.at[b2], agg_sh.at[dstb.at[1 - p]],
                                  ssems[b2]).wait()
        if issue_dst:
            pltpu.async_copy(dst_src(j + 1), dstb.at[1 - p], dsems[1 - p])
        if issue_gather:
            pltpu.async_copy(gather_src(j + 2), hbuf.at[b2], gsems[b2])

    step(0, 0, 0, False, True, True, False, True)
    step(1, 1, 1, True, True, True, True, True)

    @pl.loop(2, 122, step=6)
    def _(g):
        for o in range(6):
            step(g + o, (2 + o) % 3, o % 2, True, False, True, True, True)

    step(122, 2, 0, True, False, True, True, True)
    step(123, 0, 1, True, False, True, True, False)
    step(124, 1, 0, True, False, False, False, False)

    pltpu.make_async_copy(hbuf.at[1], agg_sh.at[dstb.at[0]], ssems[1]).wait()

    plsc.subcore_barrier()

    @pl.when(s < NSUB - 1)
    def _():
        pltpu.sync_copy(agg_sh.at[pl.ds(s * RPS, RPS)],
                        out_hbm.at[c, pl.ds(s * RPS, RPS)])

    @pl.when(s == NSUB - 1)
    def _():
        pltpu.sync_copy(agg_sh.at[pl.ds((NSUB - 1) * RPS, N - (NSUB - 1) * RPS)],
                        out_hbm.at[c, pl.ds((NSUB - 1) * RPS, N - (NSUB - 1) * RPS)])


def _sc_aggregate(eh2, src3, dst3, h2):
    mesh = plsc.VectorSubcoreMesh(core_axis_name="core",
                                  subcore_axis_name="subcore")
    kern = pl.kernel(
        _sc_body,
        out_type=jax.ShapeDtypeStruct((NSC, N, DH), jnp.float32),
        mesh=mesh,
        compiler_params=pltpu.CompilerParams(use_tc_tiling_on_sc=False),
        scratch_types=[
            pltpu.VMEM_SHARED((N, DH), jnp.float32),
            pltpu.VMEM((NCHUNK - 2, K), jnp.int32),
            pltpu.VMEM((2, K), jnp.int32),
            pltpu.VMEM((3, K, DH), jnp.float32),
            pltpu.VMEM((K, DH), jnp.float32),
            pltpu.SemaphoreType.DMA,
            pltpu.SemaphoreType.DMA,
            pltpu.SemaphoreType.DMA,
            pltpu.SemaphoreType.DMA,
            pltpu.SemaphoreType.DMA,
            pltpu.SemaphoreType.DMA,
            pltpu.SemaphoreType.DMA,
            pltpu.SemaphoreType.DMA,
            pltpu.SemaphoreType.DMA,
        ],
    )
    return kern(eh2, src3, dst3, h2)


def kernel(h_neigh, h_self, edge_attr, W_edge, W_self, W_neigh, edge_index):
    src3 = edge_index[0].astype(jnp.int32).reshape(NSUB, NCHUNK, K)
    dst3 = edge_index[1].astype(jnp.int32).reshape(NSUB, NCHUNK, K)
    eh2, h2 = _edge_mlp(edge_attr.T, W_edge, h_neigh)
    agg2 = _sc_aggregate(eh2, src3, dst3, h2)
    return _combine(h_self, W_self, W_neigh, agg2)


# speedup vs baseline: 1.1011x; 1.1011x over previous
import functools

import jax
import jax.numpy as jnp
from jax import lax
from jax.experimental import pallas as pl
from jax.experimental.pallas import tpu as pltpu
from jax.experimental.pallas import tpu_sc as plsc

N = 10000
E = 160000
D = 256
DE = 16
DH = D // 2
NSC = 2
NSUB = 16
EPS = E // NSUB
K = 80
KH = K // 2
NCHUNK = EPS // K
RPS = 624
LANES = 16


def _edge_mlp_block(ea_ref, we_ref, h_ref, out_ref, h2_ref):
    z = jax.nn.relu(lax.dot_general(
        ea_ref[...], we_ref[...],
        dimension_numbers=(((0,), (0,)), ((), ())),
        preferred_element_type=jnp.float32))
    ss = jnp.sum(z * z, axis=1, keepdims=True)
    inv = jnp.where(ss == 0.0, 1.0, lax.rsqrt(ss))
    z = z * inv
    out_ref[0] = z[:, :DH]
    out_ref[1] = z[:, DH:]
    h2_ref[0] = h_ref[:, :DH]
    h2_ref[1] = h_ref[:, DH:]


def _edge_mlp(edge_attr_t, W_edge, h_neigh):
    BE = 3200
    BN = 200
    return pl.pallas_call(
        _edge_mlp_block,
        grid=(E // BE,),
        in_specs=[
            pl.BlockSpec((DE, BE), lambda i: (0, i)),
            pl.BlockSpec((DE, D), lambda i: (0, 0)),
            pl.BlockSpec((BN, D), lambda i: (i, 0)),
        ],
        out_specs=[
            pl.BlockSpec((NSC, BE, DH), lambda i: (0, i, 0)),
            pl.BlockSpec((NSC, BN, DH), lambda i: (0, i, 0)),
        ],
        out_shape=[
            jax.ShapeDtypeStruct((NSC, E, DH), jnp.float32),
            jax.ShapeDtypeStruct((NSC, N, DH), jnp.float32),
        ],
    )(edge_attr_t, W_edge, h_neigh)


def _combine_block(hs_ref, ws_ref, wn_ref, agg_ref, out_ref):
    agg = jnp.concatenate([agg_ref[0], agg_ref[1]], axis=1)
    z = jnp.dot(hs_ref[...], ws_ref[...], preferred_element_type=jnp.float32)
    z = z + jnp.dot(agg, wn_ref[...], preferred_element_type=jnp.float32)
    z = jax.nn.relu(z)
    ss = jnp.sum(z * z, axis=1, keepdims=True)
    inv = jnp.where(ss == 0.0, 1.0, lax.rsqrt(ss))
    out_ref[...] = z * inv


def _combine(h_self, W_self, W_neigh, agg2):
    BN = 1000
    return pl.pallas_call(
        _combine_block,
        grid=(N // BN,),
        in_specs=[
            pl.BlockSpec((BN, D), lambda i: (i, 0)),
            pl.BlockSpec((D, D), lambda i: (0, 0)),
            pl.BlockSpec((D, D), lambda i: (0, 0)),
            pl.BlockSpec((NSC, BN, DH), lambda i: (0, i, 0)),
        ],
        out_specs=pl.BlockSpec((BN, D), lambda i: (i, 0)),
        out_shape=jax.ShapeDtypeStruct((N, D), jnp.float32),
    )(h_self, W_self, W_neigh, agg2)


def _sc_body(eh_hbm, src_hbm, dst_hbm, h2_hbm, out_hbm,
             agg_sh, src_v, dst_v, hbuf, ehbuf,
             sem_e, sem_g0, sem_g1):
    c = lax.axis_index("core")
    s = lax.axis_index("subcore")

    @pl.loop(0, K)
    def _(i):
        for q in range(DH // LANES):
            ehbuf[i, pl.ds(q * LANES, LANES)] = jnp.zeros((LANES,), jnp.float32)

    @pl.loop(0, 7)
    def _(r):
        pltpu.sync_copy(ehbuf, agg_sh.at[pl.ds(s * RPS + r * K, K)])

    @pl.when(s < NSUB - 1)
    def _():
        pltpu.sync_copy(ehbuf.at[pl.ds(0, 64)],
                        agg_sh.at[pl.ds(s * RPS + 7 * K, 64)])

    @pl.when(s == NSUB - 1)
    def _():
        pltpu.sync_copy(ehbuf, agg_sh.at[pl.ds(s * RPS + 7 * K, K)])

    plsc.subcore_barrier()

    pltpu.sync_copy(src_hbm.at[s], src_v)
    pltpu.sync_copy(dst_hbm.at[s], dst_v)

    sems = (sem_g0, sem_g1)

    def eh_src(j):
        return eh_hbm.at[c, pl.ds(s * EPS + j * K, K)]

    def gather_src(j):
        return h2_hbm.at[c].at[src_v.at[j]]

    pltpu.async_copy(eh_src(0), ehbuf, sem_e)
    pltpu.async_copy(gather_src(0), hbuf.at[0], sem_g0)
    pltpu.async_copy(gather_src(1), hbuf.at[1], sem_g1)

    def step(j, b, issue_eh, issue_gather):
        pltpu.make_async_copy(gather_src(j), hbuf.at[b], sems[b]).wait()
        pltpu.make_async_copy(eh_src(j), ehbuf, sem_e).wait()

        hb = hbuf.at[b]

        @pl.loop(0, K, step=4)
        def _(i):
            for r in range(4):
                for q in range(DH // LANES):
                    sl = (i + r, pl.ds(q * LANES, LANES))
                    hb[sl] = hb[sl] * ehbuf[sl]

        if issue_eh:
            pltpu.async_copy(eh_src(j + 1), ehbuf, sem_e)

        pltpu.sync_copy(hbuf.at[b], agg_sh.at[dst_v.at[j]], add=True)

        if issue_gather == "always":
            pltpu.async_copy(gather_src(j + 2), hbuf.at[b], sems[b])
        elif issue_gather == "guard":
            @pl.when(j + 2 < NCHUNK)
            def _():
                pltpu.async_copy(gather_src(j + 2), hbuf.at[b], sems[b])

    @pl.loop(0, NCHUNK - 1, step=2)
    def _(g):
        step(g, 0, True, "always")
        step(g + 1, 1, True, "guard")

    step(NCHUNK - 1, 0, False, "none")

    plsc.subcore_barrier()

    @pl.when(s < NSUB - 1)
    def _():
        pltpu.sync_copy(agg_sh.at[pl.ds(s * RPS, RPS)],
                        out_hbm.at[c, pl.ds(s * RPS, RPS)])

    @pl.when(s == NSUB - 1)
    def _():
        pltpu.sync_copy(agg_sh.at[pl.ds((NSUB - 1) * RPS, N - (NSUB - 1) * RPS)],
                        out_hbm.at[c, pl.ds((NSUB - 1) * RPS, N - (NSUB - 1) * RPS)])


def _sc_aggregate(eh2, src3, dst3, h2):
    mesh = plsc.VectorSubcoreMesh(core_axis_name="core",
                                  subcore_axis_name="subcore")
    kern = pl.kernel(
        _sc_body,
        out_type=jax.ShapeDtypeStruct((NSC, N, DH), jnp.float32),
        mesh=mesh,
        compiler_params=pltpu.CompilerParams(use_tc_tiling_on_sc=False),
        scratch_types=[
            pltpu.VMEM_SHARED((N, DH), jnp.float32),
            pltpu.VMEM((NCHUNK, K), jnp.int32),
            pltpu.VMEM((NCHUNK, K), jnp.int32),
            pltpu.VMEM((2, K, DH), jnp.float32),
            pltpu.VMEM((K, DH), jnp.float32),
            pltpu.SemaphoreType.DMA,
            pltpu.SemaphoreType.DMA,
            pltpu.SemaphoreType.DMA,
        ],
    )
    return kern(eh2, src3, dst3, h2)


def kernel(h_neigh, h_self, edge_attr, W_edge, W_self, W_neigh, edge_index):
    src3 = edge_index[0].astype(jnp.int32).reshape(NSUB, NCHUNK, K)
    dst3 = edge_index[1].astype(jnp.int32).reshape(NSUB, NCHUNK, K)
    eh2, h2 = _edge_mlp(edge_attr.T, W_edge, h_neigh)
    agg2 = _sc_aggregate(eh2, src3, dst3, h2)
    return _combine(h_self, W_self, W_neigh, agg2)


# speedup vs baseline: 1.3632x; 1.2381x over previous
import functools

import jax
import jax.numpy as jnp
from jax import lax
from jax.experimental import pallas as pl
from jax.experimental.pallas import tpu as pltpu
from jax.experimental.pallas import tpu_sc as plsc

N = 10000
E = 160000
D = 256
DE = 16
DH = D // 2
NSC = 2
NSUB = 16
EPS = E // NSUB
K = 80
KH = K // 2
NCHUNK = EPS // K
RPS = 624
LANES = 16


def _edge_mlp_block(ea_ref, we_ref, h_ref, out_ref, h2_ref):
    z = jax.nn.relu(lax.dot_general(
        ea_ref[...], we_ref[...],
        dimension_numbers=(((0,), (0,)), ((), ())),
        preferred_element_type=jnp.float32))
    ss = jnp.sum(z * z, axis=1, keepdims=True)
    inv = jnp.where(ss == 0.0, 1.0, lax.rsqrt(ss))
    z = z * inv
    out_ref[0] = z[:, :DH]
    out_ref[1] = z[:, DH:]
    h2_ref[0] = h_ref[:, :DH]
    h2_ref[1] = h_ref[:, DH:]


def _edge_mlp(edge_attr_t, W_edge, h_neigh):
    BE = 3200
    BN = 200
    return pl.pallas_call(
        _edge_mlp_block,
        grid=(E // BE,),
        in_specs=[
            pl.BlockSpec((DE, BE), lambda i: (0, i)),
            pl.BlockSpec((DE, D), lambda i: (0, 0)),
            pl.BlockSpec((BN, D), lambda i: (i, 0)),
        ],
        out_specs=[
            pl.BlockSpec((NSC, BE, DH), lambda i: (0, i, 0)),
            pl.BlockSpec((NSC, BN, DH), lambda i: (0, i, 0)),
        ],
        out_shape=[
            jax.ShapeDtypeStruct((NSC, E, DH), jnp.float32),
            jax.ShapeDtypeStruct((NSC, N, DH), jnp.float32),
        ],
    )(edge_attr_t, W_edge, h_neigh)


def _combine_block(hs_ref, ws_ref, wn_ref, agg_ref, out_ref):
    agg = jnp.concatenate([agg_ref[0], agg_ref[1]], axis=1)
    z = jnp.dot(hs_ref[...], ws_ref[...], preferred_element_type=jnp.float32)
    z = z + jnp.dot(agg, wn_ref[...], preferred_element_type=jnp.float32)
    z = jax.nn.relu(z)
    ss = jnp.sum(z * z, axis=1, keepdims=True)
    inv = jnp.where(ss == 0.0, 1.0, lax.rsqrt(ss))
    out_ref[...] = z * inv


def _combine(h_self, W_self, W_neigh, agg2):
    BN = 1000
    return pl.pallas_call(
        _combine_block,
        grid=(N // BN,),
        in_specs=[
            pl.BlockSpec((BN, D), lambda i: (i, 0)),
            pl.BlockSpec((D, D), lambda i: (0, 0)),
            pl.BlockSpec((D, D), lambda i: (0, 0)),
            pl.BlockSpec((NSC, BN, DH), lambda i: (0, i, 0)),
        ],
        out_specs=pl.BlockSpec((BN, D), lambda i: (i, 0)),
        out_shape=jax.ShapeDtypeStruct((N, D), jnp.float32),
    )(h_self, W_self, W_neigh, agg2)


def _sc_body(eh_hbm, src_hbm, dst_hbm, h2_hbm, out_hbm,
             agg_sh, src_v, dstb, hbuf, ehb,
             sem_e0, sem_e1, sem_g0, sem_g1, sem_d0, sem_d1):
    c = lax.axis_index("core")
    s = lax.axis_index("subcore")

    zb = ehb.at[0]

    @pl.loop(0, K)
    def _(i):
        for q in range(DH // LANES):
            zb[i, pl.ds(q * LANES, LANES)] = jnp.zeros((LANES,), jnp.float32)

    @pl.loop(0, 7)
    def _(r):
        pltpu.sync_copy(zb, agg_sh.at[pl.ds(s * RPS + r * K, K)])

    @pl.when(s < NSUB - 1)
    def _():
        pltpu.sync_copy(zb.at[pl.ds(0, 64)],
                        agg_sh.at[pl.ds(s * RPS + 7 * K, 64)])

    @pl.when(s == NSUB - 1)
    def _():
        pltpu.sync_copy(zb, agg_sh.at[pl.ds(s * RPS + 7 * K, K)])

    plsc.subcore_barrier()

    pltpu.sync_copy(src_hbm.at[s, pl.ds(2, NCHUNK - 2)], src_v)

    esems = (sem_e0, sem_e1)
    gsems = (sem_g0, sem_g1)
    dsems = (sem_d0, sem_d1)

    def eh_src(j):
        return eh_hbm.at[c, pl.ds(s * EPS + j * K, K)]

    def gather_src(j):
        return h2_hbm.at[c].at[src_v.at[j - 2]]

    def dst_src(j):
        return dst_hbm.at[s, j]

    pltpu.sync_copy(src_hbm.at[s, 0], dstb.at[0])
    pltpu.sync_copy(src_hbm.at[s, 1], dstb.at[1])
    pltpu.async_copy(eh_src(0), ehb.at[0], sem_e0)
    pltpu.async_copy(eh_src(1), ehb.at[1], sem_e1)
    pltpu.async_copy(h2_hbm.at[c].at[dstb.at[0]], hbuf.at[0], sem_g0)
    pltpu.async_copy(h2_hbm.at[c].at[dstb.at[1]], hbuf.at[1], sem_g1)

    def step(j, b, dst_sync, issue_eh, issue_dst, issue_gather):
        if dst_sync:
            pltpu.make_async_copy(h2_hbm.at[c].at[dstb.at[b]],
                                  hbuf.at[b], gsems[b]).wait()
            pltpu.sync_copy(dst_src(j), dstb.at[b])
        else:
            pltpu.make_async_copy(gather_src(j), hbuf.at[b],
                                  gsems[b]).wait()
        pltpu.make_async_copy(eh_src(j), ehb.at[b], esems[b]).wait()

        hb = hbuf.at[b]
        eb = ehb.at[b]

        @pl.loop(0, K, step=4)
        def _(i):
            for r in range(4):
                for q in range(DH // LANES):
                    sl = (i + r, pl.ds(q * LANES, LANES))
                    hb[sl] = hb[sl] * eb[sl]

        if issue_eh:
            pltpu.async_copy(eh_src(j + 2), ehb.at[b], esems[b])

        if not dst_sync:
            pltpu.make_async_copy(dst_src(j), dstb.at[b], dsems[b]).wait()

        pltpu.sync_copy(hbuf.at[b], agg_sh.at[dstb.at[b]], add=True)

        if issue_dst:
            pltpu.async_copy(dst_src(j + 1), dstb.at[1 - b], dsems[1 - b])
        if issue_gather:
            pltpu.async_copy(gather_src(j + 2), hbuf.at[b], gsems[b])

    step(0, 0, True, True, False, True)
    step(1, 1, True, True, True, True)

    @pl.loop(2, 122, step=2)
    def _(g):
        step(g, 0, False, True, True, True)
        step(g + 1, 1, False, True, True, True)

    step(122, 0, False, True, True, True)
    step(123, 1, False, False, True, False)
    step(124, 0, False, False, False, False)

    plsc.subcore_barrier()

    @pl.when(s < NSUB - 1)
    def _():
        pltpu.sync_copy(agg_sh.at[pl.ds(s * RPS, RPS)],
                        out_hbm.at[c, pl.ds(s * RPS, RPS)])

    @pl.when(s == NSUB - 1)
    def _():
        pltpu.sync_copy(agg_sh.at[pl.ds((NSUB - 1) * RPS, N - (NSUB - 1) * RPS)],
                        out_hbm.at[c, pl.ds((NSUB - 1) * RPS, N - (NSUB - 1) * RPS)])


def _sc_aggregate(eh2, src3, dst3, h2):
    mesh = plsc.VectorSubcoreMesh(core_axis_name="core",
                                  subcore_axis_name="subcore")
    kern = pl.kernel(
        _sc_body,
        out_type=jax.ShapeDtypeStruct((NSC, N, DH), jnp.float32),
        mesh=mesh,
        compiler_params=pltpu.CompilerParams(use_tc_tiling_on_sc=False),
        scratch_types=[
            pltpu.VMEM_SHARED((N, DH), jnp.float32),
            pltpu.VMEM((NCHUNK - 2, K), jnp.int32),
            pltpu.VMEM((2, K), jnp.int32),
            pltpu.VMEM((2, K, DH), jnp.float32),
            pltpu.VMEM((2, K, DH), jnp.float32),
            pltpu.SemaphoreType.DMA,
            pltpu.SemaphoreType.DMA,
            pltpu.SemaphoreType.DMA,
            pltpu.SemaphoreType.DMA,
            pltpu.SemaphoreType.DMA,
            pltpu.SemaphoreType.DMA,
        ],
    )
    return kern(eh2, src3, dst3, h2)


def kernel(h_neigh, h_self, edge_attr, W_edge, W_self, W_neigh, edge_index):
    src3 = edge_index[0].astype(jnp.int32).reshape(NSUB, NCHUNK, K)
    dst3 = edge_index[1].astype(jnp.int32).reshape(NSUB, NCHUNK, K)
    eh2, h2 = _edge_mlp(edge_attr.T, W_edge, h_neigh)
    agg2 = _sc_aggregate(eh2, src3, dst3, h2)
    return _combine(h_self, W_self, W_neigh, agg2)
